# baseline (speedup 1.0000x reference)
import functools

import jax
import jax.numpy as jnp
from jax import lax
from jax.experimental import pallas as pl
from jax.experimental.pallas import tpu as pltpu
from jax.experimental.pallas import tpu_sc as plsc

NUM_EMB = 1000000
DIM = 32
BATCH = 16384
HIST = 20
TOTAL = BATCH * HIST

_info = plsc.get_sparse_core_info()
_NC = _info.num_cores
_NS = _info.num_subcores
_NW = _NC * _NS
_L = _info.num_lanes

_NTC = NUM_EMB // 128
_TC_LAST = NUM_EMB - _NTC * 128


def _relayout_body(wt_hbm, tail_hbm, out_hbm, buf, obuf, in_sems, out_sems, tail_sem):
    wid = lax.axis_index("s") * _NC + lax.axis_index("c")
    n_i = lax.select(wid < _NTC % _NW, _NTC // _NW + 1, _NTC // _NW)
    iota = lax.iota(jnp.int32, _L)

    def in_copy(i):
        jt = wid + i * _NW
        b = lax.rem(i, 2)
        return pltpu.make_async_copy(
            wt_hbm.at[:, pl.ds(jt * 128, 128)],
            buf.at[b, :, pl.ds(0, 128)],
            in_sems.at[b],
        )

    def out_copy(i):
        jt = wid + i * _NW
        b = lax.rem(i, 2)
        return pltpu.make_async_copy(
            obuf.at[b],
            out_hbm.at[pl.ds(jt * 32, 32), :],
            out_sems.at[b],
        )

    def transpose(i):
        b = lax.rem(i, 2)

        def row(k, _):
            for m0 in range(0, 128, _L):
                s = m0 // 32
                d0 = m0 % 32
                col = 4 * k + s
                vec = plsc.load_gather(
                    buf, [jnp.broadcast_to(b, (_L,)), d0 + iota,
                          jnp.broadcast_to(col, (_L,))]
                )
                obuf[b, k, pl.ds(m0, _L)] = vec
            return _

        lax.fori_loop(0, 32, row, 0, unroll=2)

    in_copy(0).start()

    def step(i, carry):
        in_copy(i).wait()

        @pl.when(i + 1 < n_i)
        def _():
            in_copy(i + 1).start()

        @pl.when(i >= 2)
        def _():
            out_copy(i - 2).wait()

        transpose(i)
        out_copy(i).start()
        return carry

    lax.fori_loop(0, n_i, step, 0)

    @pl.when(n_i >= 2)
    def _():
        out_copy(n_i - 2).wait()

    @pl.when(n_i >= 1)
    def _():
        out_copy(n_i - 1).wait()

    @pl.when(wid == _NW - 1)
    def _():
        pltpu.async_copy(
            tail_hbm, obuf.at[0, pl.ds(0, _TC_LAST // 4), :], tail_sem
        ).wait()
        pltpu.async_copy(
            obuf.at[0, pl.ds(0, _TC_LAST // 4), :],
            out_hbm.at[pl.ds(_NTC * 32, _TC_LAST // 4), :],
            tail_sem,
        ).wait()


_w_relayout = pl.kernel(
    _relayout_body,
    out_type=jax.ShapeDtypeStruct((NUM_EMB // 4, 128), jnp.float32),
    mesh=plsc.VectorSubcoreMesh(core_axis_name="c", subcore_axis_name="s"),
    scratch_types=[
        pltpu.VMEM((2, DIM, 129), jnp.float32),
        pltpu.VMEM((2, DIM, 128), jnp.float32),
        pltpu.SemaphoreType.DMA((2,)),
        pltpu.SemaphoreType.DMA((2,)),
        pltpu.SemaphoreType.DMA,
    ],
    compiler_params=pltpu.CompilerParams(
        use_tc_tiling_on_sc=True, needs_layout_passes=False
    ),
)

_B_PER_W = TOTAL // _NW
_CHUNK = 512
_NCHUNKS = _B_PER_W // _CHUNK
_GBUF = 3
_TBUF = 2
_TSTR = _CHUNK + 1


def _body(idx_hbm, table_hbm, out_hbm, idx_v, rows_v, t_v, *sems):
    g_sems = sems[:_GBUF]
    o_sems = sems[_GBUF:]
    wid = lax.axis_index("s") * _NC + lax.axis_index("c")
    base = wid * _B_PER_W
    pltpu.sync_copy(idx_hbm.at[pl.ds(base, _B_PER_W)], idx_v)

    iota = lax.iota(jnp.int32, _L)
    d_lo = iota
    d_hi = iota + _L

    def start_gather(c):
        g = c % _GBUF
        idx_sl = idx_v.at[pl.ds(c * _CHUNK, _CHUNK)]
        return pltpu.async_copy(table_hbm.at[idx_sl], rows_v.at[g], g_sems[g])

    def start_out(c):
        t = c % _TBUF
        j0 = base + c * _CHUNK
        h = j0 // BATCH
        b0 = j0 % BATCH
        return pltpu.async_copy(
            t_v.at[t, :, pl.ds(0, _CHUNK)],
            out_hbm.at[h, :, pl.ds(b0, _CHUNK)],
            o_sems[t],
        )

    gathers = [start_gather(c) for c in range(min(_GBUF, _NCHUNKS))]
    gathers += [None] * (_NCHUNKS - len(gathers))
    outs = [None] * _NCHUNKS
    for c in range(_NCHUNKS):
        g = c % _GBUF
        t = c % _TBUF
        gathers[c].wait()
        if c >= _TBUF:
            outs[c - _TBUF].wait()

        def transpose_one(l, _, g=g, t=t):
            x0 = rows_v[g, l, pl.ds(0, _L)]
            x1 = rows_v[g, l, pl.ds(_L, _L)]
            lv = jnp.broadcast_to(l, (_L,))
            tv = jnp.broadcast_to(t, (_L,))
            plsc.store_scatter(t_v, [tv, d_lo, lv], x0)
            plsc.store_scatter(t_v, [tv, d_hi, lv], x1)
            return _

        lax.fori_loop(0, _CHUNK, transpose_one, 0, unroll=8)
        outs[c] = start_out(c)
        nc = c + _GBUF
        if nc < _NCHUNKS:
            gathers[nc] = start_gather(nc)
    for c in range(_NCHUNKS - _TBUF, _NCHUNKS):
        outs[c].wait()


_gather = pl.kernel(
    _body,
    out_type=jax.ShapeDtypeStruct((HIST, DIM, BATCH), jnp.float32),
    mesh=plsc.VectorSubcoreMesh(core_axis_name="c", subcore_axis_name="s"),
    scratch_types=[
        pltpu.VMEM((_B_PER_W,), jnp.int32),
        pltpu.VMEM((_GBUF, _CHUNK, DIM), jnp.float32),
        pltpu.VMEM((_TBUF, DIM, _TSTR), jnp.float32),
    ]
    + [pltpu.SemaphoreType.DMA] * (_GBUF + _TBUF),
    compiler_params=pltpu.CompilerParams(
        use_tc_tiling_on_sc=False, needs_layout_passes=False
    ),
)


@jax.jit
def kernel(token_ids, weights):
    idx = jnp.reshape(token_ids.T, (TOTAL,)).astype(jnp.int32)
    tail = jnp.reshape(weights[_NTC * 128:], (_TC_LAST // 4, 128))
    w128 = _w_relayout(weights.T, tail)
    w_lin = jnp.reshape(w128, (NUM_EMB, DIM))
    out_t = _gather(idx, w_lin)
    return jnp.transpose(out_t, (2, 0, 1))


# speedup vs baseline: 1.2127x; 1.2127x over previous
import functools

import jax
import jax.numpy as jnp
from jax import lax
from jax.experimental import pallas as pl
from jax.experimental.pallas import tpu as pltpu
from jax.experimental.pallas import tpu_sc as plsc

NUM_EMB = 1000000
DIM = 32
BATCH = 16384
HIST = 20
TOTAL = BATCH * HIST

_info = plsc.get_sparse_core_info()
_NC = _info.num_cores
_NS = _info.num_subcores
_NW = _NC * _NS
_L = _info.num_lanes

_NTC = NUM_EMB // 128
_TC_LAST = NUM_EMB - _NTC * 128


def _slab_body(wt_hbm, out_hbm, buf, in_sems, out_sems):
    wid = lax.axis_index("s") * _NC + lax.axis_index("c")
    n_i = lax.select(wid < _NTC % _NW, _NTC // _NW + 1, _NTC // _NW)

    def s_in(i):
        jt = wid + i * _NW
        b = lax.rem(i, 2)
        return pltpu.make_async_copy(
            wt_hbm.at[:, pl.ds(jt * 128, 128)], buf.at[b], in_sems.at[b]
        )

    def s_out(i):
        jt = wid + i * _NW
        b = lax.rem(i, 2)
        return pltpu.make_async_copy(buf.at[b], out_hbm.at[jt], out_sems.at[b])

    s_in(0).start()

    def step(i, carry):
        s_in(i).wait()

        @pl.when(i + 1 < n_i)
        def _():
            s_in(i + 1).start()

        @pl.when(i >= 2)
        def _():
            s_out(i - 2).wait()

        s_out(i).start()
        return carry

    lax.fori_loop(0, n_i, step, 0)

    @pl.when(n_i >= 2)
    def _():
        s_out(n_i - 2).wait()

    @pl.when(n_i >= 1)
    def _():
        s_out(n_i - 1).wait()


_w_slabs = pl.kernel(
    _slab_body,
    out_type=jax.ShapeDtypeStruct((_NTC, DIM, 128), jnp.float32),
    mesh=plsc.VectorSubcoreMesh(core_axis_name="c", subcore_axis_name="s"),
    scratch_types=[
        pltpu.VMEM((2, DIM, 128), jnp.float32),
        pltpu.SemaphoreType.DMA((2,)),
        pltpu.SemaphoreType.DMA((2,)),
    ],
    compiler_params=pltpu.CompilerParams(
        use_tc_tiling_on_sc=True, needs_layout_passes=False
    ),
)


def _relayout_body(wt_hbm, tail_hbm, out_hbm, buf, obuf, in_sems, out_sems, tail_sem):
    wid = lax.axis_index("s") * _NC + lax.axis_index("c")
    n_i = lax.select(wid < _NTC % _NW, _NTC // _NW + 1, _NTC // _NW)
    iota = lax.iota(jnp.int32, _L)

    def in_copy(i):
        jt = wid + i * _NW
        b = lax.rem(i, 2)
        return pltpu.make_async_copy(
            wt_hbm.at[jt],
            buf.at[b, :, pl.ds(0, 128)],
            in_sems.at[b],
        )

    def out_copy(i):
        jt = wid + i * _NW
        b = lax.rem(i, 2)
        return pltpu.make_async_copy(
            obuf.at[b],
            out_hbm.at[pl.ds(jt * 32, 32), :],
            out_sems.at[b],
        )

    def transpose(i):
        b = lax.rem(i, 2)

        def row(k, _):
            for m0 in range(0, 128, _L):
                s = m0 // 32
                d0 = m0 % 32
                col = 4 * k + s
                vec = plsc.load_gather(
                    buf, [jnp.broadcast_to(b, (_L,)), d0 + iota,
                          jnp.broadcast_to(col, (_L,))]
                )
                obuf[b, k, pl.ds(m0, _L)] = vec
            return _

        lax.fori_loop(0, 32, row, 0, unroll=2)

    in_copy(0).start()

    def step(i, carry):
        in_copy(i).wait()

        @pl.when(i + 1 < n_i)
        def _():
            in_copy(i + 1).start()

        @pl.when(i >= 2)
        def _():
            out_copy(i - 2).wait()

        transpose(i)
        out_copy(i).start()
        return carry

    lax.fori_loop(0, n_i, step, 0)

    @pl.when(n_i >= 2)
    def _():
        out_copy(n_i - 2).wait()

    @pl.when(n_i >= 1)
    def _():
        out_copy(n_i - 1).wait()

    @pl.when(wid == _NW - 1)
    def _():
        pltpu.async_copy(
            tail_hbm, obuf.at[0, pl.ds(0, _TC_LAST // 4), :], tail_sem
        ).wait()
        pltpu.async_copy(
            obuf.at[0, pl.ds(0, _TC_LAST // 4), :],
            out_hbm.at[pl.ds(_NTC * 32, _TC_LAST // 4), :],
            tail_sem,
        ).wait()


_w_relayout = pl.kernel(
    _relayout_body,
    out_type=jax.ShapeDtypeStruct((NUM_EMB // 4, 128), jnp.float32),
    mesh=plsc.VectorSubcoreMesh(core_axis_name="c", subcore_axis_name="s"),
    scratch_types=[
        pltpu.VMEM((2, DIM, 129), jnp.float32),
        pltpu.VMEM((2, DIM, 128), jnp.float32),
        pltpu.SemaphoreType.DMA((2,)),
        pltpu.SemaphoreType.DMA((2,)),
        pltpu.SemaphoreType.DMA,
    ],
    compiler_params=pltpu.CompilerParams(
        use_tc_tiling_on_sc=False, needs_layout_passes=False
    ),
)

_B_PER_W = TOTAL // _NW
_CHUNK = 512
_NCHUNKS = _B_PER_W // _CHUNK
_GBUF = 3
_TBUF = 2
_TSTR = _CHUNK + 1


def _body(idx_hbm, table_hbm, out_hbm, idx_v, rows_v, t_v, *sems):
    g_sems = sems[:_GBUF]
    o_sems = sems[_GBUF:]
    wid = lax.axis_index("s") * _NC + lax.axis_index("c")
    base = wid * _B_PER_W
    pltpu.sync_copy(idx_hbm.at[pl.ds(base, _B_PER_W)], idx_v)

    iota = lax.iota(jnp.int32, _L)
    d_lo = iota
    d_hi = iota + _L

    def start_gather(c):
        g = c % _GBUF
        idx_sl = idx_v.at[pl.ds(c * _CHUNK, _CHUNK)]
        return pltpu.async_copy(table_hbm.at[idx_sl], rows_v.at[g], g_sems[g])

    def start_out(c):
        t = c % _TBUF
        j0 = base + c * _CHUNK
        h = j0 // BATCH
        b0 = j0 % BATCH
        return pltpu.async_copy(
            t_v.at[t, :, pl.ds(0, _CHUNK)],
            out_hbm.at[h, :, pl.ds(b0, _CHUNK)],
            o_sems[t],
        )

    gathers = [start_gather(c) for c in range(min(_GBUF, _NCHUNKS))]
    gathers += [None] * (_NCHUNKS - len(gathers))
    outs = [None] * _NCHUNKS
    for c in range(_NCHUNKS):
        g = c % _GBUF
        t = c % _TBUF
        gathers[c].wait()
        if c >= _TBUF:
            outs[c - _TBUF].wait()

        def transpose_one(l, _, g=g, t=t):
            x0 = rows_v[g, l, pl.ds(0, _L)]
            x1 = rows_v[g, l, pl.ds(_L, _L)]
            lv = jnp.broadcast_to(l, (_L,))
            tv = jnp.broadcast_to(t, (_L,))
            plsc.store_scatter(t_v, [tv, d_lo, lv], x0)
            plsc.store_scatter(t_v, [tv, d_hi, lv], x1)
            return _

        lax.fori_loop(0, _CHUNK, transpose_one, 0, unroll=8)
        outs[c] = start_out(c)
        nc = c + _GBUF
        if nc < _NCHUNKS:
            gathers[nc] = start_gather(nc)
    for c in range(_NCHUNKS - _TBUF, _NCHUNKS):
        outs[c].wait()


_gather = pl.kernel(
    _body,
    out_type=jax.ShapeDtypeStruct((HIST, DIM, BATCH), jnp.float32),
    mesh=plsc.VectorSubcoreMesh(core_axis_name="c", subcore_axis_name="s"),
    scratch_types=[
        pltpu.VMEM((_B_PER_W,), jnp.int32),
        pltpu.VMEM((_GBUF, _CHUNK, DIM), jnp.float32),
        pltpu.VMEM((_TBUF, DIM, _TSTR), jnp.float32),
    ]
    + [pltpu.SemaphoreType.DMA] * (_GBUF + _TBUF),
    compiler_params=pltpu.CompilerParams(
        use_tc_tiling_on_sc=False, needs_layout_passes=False
    ),
)


@jax.jit
def kernel(token_ids, weights):
    idx = jnp.reshape(token_ids.T, (TOTAL,)).astype(jnp.int32)
    tail = jnp.reshape(weights[_NTC * 128:], (_TC_LAST // 4, 128))
    slabs = _w_slabs(weights.T)
    w128 = _w_relayout(slabs, tail)
    w_lin = jnp.reshape(w128, (NUM_EMB, DIM))
    out_t = _gather(idx, w_lin)
    return jnp.transpose(out_t, (2, 0, 1))


# speedup vs baseline: 1.5511x; 1.2791x over previous
import functools

import jax
import jax.numpy as jnp
from jax import lax
from jax.experimental import pallas as pl
from jax.experimental.pallas import tpu as pltpu
from jax.experimental.pallas import tpu_sc as plsc

NUM_EMB = 1000000
DIM = 32
BATCH = 16384
HIST = 20
TOTAL = BATCH * HIST

_info = plsc.get_sparse_core_info()
_NC = _info.num_cores
_NS = _info.num_subcores
_NW = _NC * _NS
_L = _info.num_lanes

_B_PER_W = TOTAL // _NW
_CHUNK = 512
_NCHUNKS = _B_PER_W // _CHUNK
_GBUF = 3
_TBUF = 2
_NRT = DIM // 8
_NCT = _CHUNK // 128
_RPAD = 10


def _body(idx_hbm, table_hbm, out_hbm, idx_v, rows_v, t_v, *sems):
    g_sems = sems[:_GBUF]
    o_sems = sems[_GBUF:]
    wid = lax.axis_index("s") * _NC + lax.axis_index("c")
    base = wid * _B_PER_W
    pltpu.sync_copy(idx_hbm.at[pl.ds(base, _B_PER_W)], idx_v)

    iota = lax.iota(jnp.int32, _L)
    ri_lo = iota // 8
    r_lo = lax.rem(iota, 8)
    ri_hi = ri_lo + 2

    def start_gather(c):
        g = c % _GBUF
        idx_sl = idx_v.at[pl.ds(c * _CHUNK, _CHUNK)]
        return pltpu.async_copy(table_hbm.at[idx_sl], rows_v.at[g], g_sems[g])

    def start_out(c):
        t = c % _TBUF
        j0 = base + c * _CHUNK
        h = j0 // BATCH
        cj0 = (j0 % BATCH) // 128
        return pltpu.async_copy(
            t_v.at[t, :, :, pl.ds(0, 8), pl.ds(0, 128)],
            out_hbm.at[h, :, pl.ds(cj0, _NCT), :, :],
            o_sems[t],
        )

    gathers = [start_gather(c) for c in range(min(_GBUF, _NCHUNKS))]
    gathers += [None] * (_NCHUNKS - len(gathers))
    outs = [None] * _NCHUNKS
    for c in range(_NCHUNKS):
        g = c % _GBUF
        t = c % _TBUF
        gathers[c].wait()
        if c >= _TBUF:
            outs[c - _TBUF].wait()

        def transpose_one(l, carry, g=g, t=t):
            x0 = rows_v[g, l, pl.ds(0, _L)]
            x1 = rows_v[g, l, pl.ds(_L, _L)]
            cj = l // 128
            cc = lax.rem(l, 128)
            tv = jnp.broadcast_to(t, (_L,))
            cjv = jnp.broadcast_to(cj, (_L,))
            ccv = jnp.broadcast_to(cc, (_L,))
            plsc.store_scatter(t_v, [tv, ri_lo, cjv, r_lo, ccv], x0)
            plsc.store_scatter(t_v, [tv, ri_hi, cjv, r_lo, ccv], x1)
            return carry

        lax.fori_loop(0, _CHUNK, transpose_one, 0, unroll=8)
        outs[c] = start_out(c)
        nc = c + _GBUF
        if nc < _NCHUNKS:
            gathers[nc] = start_gather(nc)
    for c in range(_NCHUNKS - _TBUF, _NCHUNKS):
        outs[c].wait()


_gather = pl.kernel(
    _body,
    out_type=jax.ShapeDtypeStruct((HIST, _NRT, BATCH // 128, 8, 128), jnp.float32),
    mesh=plsc.VectorSubcoreMesh(core_axis_name="c", subcore_axis_name="s"),
    scratch_types=[
        pltpu.VMEM((_B_PER_W,), jnp.int32),
        pltpu.VMEM((_GBUF, _CHUNK, DIM), jnp.float32),
        pltpu.VMEM((_TBUF, _NRT, _NCT, _RPAD, 129), jnp.float32),
    ]
    + [pltpu.SemaphoreType.DMA] * (_GBUF + _TBUF),
    compiler_params=pltpu.CompilerParams(
        use_tc_tiling_on_sc=False, needs_layout_passes=False
    ),
)


@jax.jit
def kernel(token_ids, weights):
    idx = jnp.reshape(token_ids.T, (TOTAL,)).astype(jnp.int32)
    out5 = _gather(idx, weights)
    out = jnp.transpose(out5, (2, 4, 0, 1, 3))
    return jnp.reshape(out, (BATCH, HIST, DIM))


# speedup vs baseline: 1.5528x; 1.0011x over previous
import functools

import jax
import jax.numpy as jnp
from jax import lax
from jax.experimental import pallas as pl
from jax.experimental.pallas import tpu as pltpu
from jax.experimental.pallas import tpu_sc as plsc

NUM_EMB = 1000000
DIM = 32
BATCH = 16384
HIST = 20
TOTAL = BATCH * HIST

_info = plsc.get_sparse_core_info()
_NC = _info.num_cores
_NS = _info.num_subcores
_NW = _NC * _NS
_L = _info.num_lanes

_B_PER_W = TOTAL // _NW
_CHUNK = 512
_NCHUNKS = _B_PER_W // _CHUNK
_GBUF = 3
_TBUF = 3
_NRT = DIM // 8
_NCT = _CHUNK // 128
_RPAD = 10


def _body(idx_hbm, table_hbm, out_hbm, idx_v, rows_v, t_v, *sems):
    g_sems = sems[:_GBUF]
    o_sems = sems[_GBUF:]
    wid = lax.axis_index("s") * _NC + lax.axis_index("c")
    base = wid * _B_PER_W
    pltpu.sync_copy(idx_hbm.at[pl.ds(base, _B_PER_W)], idx_v)

    iota = lax.iota(jnp.int32, _L)
    ri_lo = iota // 8
    r_lo = lax.rem(iota, 8)
    ri_hi = ri_lo + 2

    def start_gather(c):
        g = c % _GBUF
        idx_sl = idx_v.at[pl.ds(c * _CHUNK, _CHUNK)]
        return pltpu.async_copy(table_hbm.at[idx_sl], rows_v.at[g], g_sems[g])

    def start_out(c):
        t = c % _TBUF
        j0 = base + c * _CHUNK
        h = j0 // BATCH
        cj0 = (j0 % BATCH) // 128
        return pltpu.async_copy(
            t_v.at[t, :, :, pl.ds(0, 8), pl.ds(0, 128)],
            out_hbm.at[h, :, pl.ds(cj0, _NCT), :, :],
            o_sems[t],
        )

    gathers = [start_gather(c) for c in range(min(_GBUF, _NCHUNKS))]
    gathers += [None] * (_NCHUNKS - len(gathers))
    outs = [None] * _NCHUNKS
    for c in range(_NCHUNKS):
        g = c % _GBUF
        t = c % _TBUF
        gathers[c].wait()
        if c >= _TBUF:
            outs[c - _TBUF].wait()

        def transpose_one(l, carry, g=g, t=t):
            x0 = rows_v[g, l, pl.ds(0, _L)]
            x1 = rows_v[g, l, pl.ds(_L, _L)]
            cj = l // 128
            cc = lax.rem(l, 128)
            tv = jnp.broadcast_to(t, (_L,))
            cjv = jnp.broadcast_to(cj, (_L,))
            ccv = jnp.broadcast_to(cc, (_L,))
            plsc.store_scatter(t_v, [tv, ri_lo, cjv, r_lo, ccv], x0)
            plsc.store_scatter(t_v, [tv, ri_hi, cjv, r_lo, ccv], x1)
            return carry

        lax.fori_loop(0, _CHUNK, transpose_one, 0, unroll=8)
        outs[c] = start_out(c)
        nc = c + _GBUF
        if nc < _NCHUNKS:
            gathers[nc] = start_gather(nc)
    for c in range(_NCHUNKS - _TBUF, _NCHUNKS):
        outs[c].wait()


_gather = pl.kernel(
    _body,
    out_type=jax.ShapeDtypeStruct((HIST, _NRT, BATCH // 128, 8, 128), jnp.float32),
    mesh=plsc.VectorSubcoreMesh(core_axis_name="c", subcore_axis_name="s"),
    scratch_types=[
        pltpu.VMEM((_B_PER_W,), jnp.int32),
        pltpu.VMEM((_GBUF, _CHUNK, DIM), jnp.float32),
        pltpu.VMEM((_TBUF, _NRT, _NCT, _RPAD, 129), jnp.float32),
    ]
    + [pltpu.SemaphoreType.DMA] * (_GBUF + _TBUF),
    compiler_params=pltpu.CompilerParams(
        use_tc_tiling_on_sc=False, needs_layout_passes=False
    ),
)


@jax.jit
def kernel(token_ids, weights):
    idx = jnp.reshape(token_ids.T, (TOTAL,)).astype(jnp.int32)
    out5 = _gather(idx, weights)
    out = jnp.transpose(out5, (2, 4, 0, 1, 3))
    return jnp.reshape(out, (BATCH, HIST, DIM))
